# baseline (device time: 12731 ns/iter reference)
import jax
import jax.numpy as jnp
from jax import lax
from jax.experimental import pallas as pl
from jax.experimental.pallas import tpu as pltpu

N_DEV = 4
TAPS = 4
HALO = TAPS - 1


def kernel(x, k):
    b, s, c = x.shape

    def body(x_ref, k_ref, out_ref, halo_ref, send_sem, recv_sem):
        my = lax.axis_index("i")
        left = (my - 1) % N_DEV
        right = (my + 1) % N_DEV

        barrier_sem = pltpu.get_barrier_semaphore()
        for nbr in [left, right]:
            pl.semaphore_signal(
                barrier_sem, inc=1,
                device_id=(nbr,), device_id_type=pl.DeviceIdType.MESH,
            )
        pl.semaphore_wait(barrier_sem, 2)

        rdma = pltpu.make_async_remote_copy(
            src_ref=x_ref.at[:, pl.ds(s - HALO, HALO), :],
            dst_ref=halo_ref,
            send_sem=send_sem,
            recv_sem=recv_sem,
            device_id=(right,),
            device_id_type=pl.DeviceIdType.MESH,
        )
        rdma.start()

        one = jnp.bfloat16(1.0)
        xv = x_ref[...].astype(jnp.bfloat16)
        kv = k_ref[...].astype(jnp.bfloat16)

        acc = xv * kv[TAPS - 1][None, None, :]
        for d in range(1, TAPS):
            acc += pltpu.roll(xv, d, axis=1) * kv[TAPS - 1 - d][None, None, :]
        out_ref[...] = acc * (one / (one + jnp.exp(-acc)))

        rdma.wait()

        halo = jnp.where(my == 0, 0.0, halo_ref[...]).astype(jnp.bfloat16)
        head = jnp.concatenate([halo, xv[:, 0:HALO, :]], axis=1)
        acc_h = head[:, 0:HALO, :] * kv[0][None, None, :]
        for t in range(1, TAPS - 1):
            acc_h += head[:, t:t + HALO, :] * kv[t][None, None, :]
        acc_h += xv[:, 0:HALO, :] * kv[TAPS - 1][None, None, :]
        out_ref[:, 0:HALO, :] = acc_h * (one / (one + jnp.exp(-acc_h)))

    return pl.pallas_call(
        body,
        out_shape=jax.ShapeDtypeStruct((b, s, c), jnp.bfloat16),
        in_specs=[
            pl.BlockSpec(memory_space=pltpu.VMEM),
            pl.BlockSpec(memory_space=pltpu.VMEM),
        ],
        out_specs=pl.BlockSpec(memory_space=pltpu.VMEM),
        scratch_shapes=[
            pltpu.VMEM((b, HALO, c), x.dtype),
            pltpu.SemaphoreType.DMA,
            pltpu.SemaphoreType.DMA,
        ],
        compiler_params=pltpu.CompilerParams(collective_id=0),
    )(x, k)
